# Initial kernel scaffold; baseline (speedup 1.0000x reference)
import functools

import jax
import jax.numpy as jnp
from jax import lax
from jax.experimental import pallas as pl
from jax.experimental.pallas import tpu as pltpu
from jax.experimental.pallas import tpu_sc as plsc

N = 10000
E = 320000
IN = 128
HID = 256
OUT = 128

NC = 2
NS = 16
CHUNK = 128
E_PAD = 323584
PAD_ROWS = 64
N_ACC = N + PAD_ROWS
DEG_STRIPE = 632
N_DEG = NS * DEG_STRIPE

_mesh = plsc.VectorSubcoreMesh(core_axis_name="c", subcore_axis_name="s")


@functools.partial(
    pl.kernel,
    out_type=jax.ShapeDtypeStruct((NC, N_DEG), jnp.float32),
    mesh=_mesh,
    scratch_types=[
        pltpu.VMEM_SHARED((N_DEG,), jnp.float32),
        pltpu.VMEM((CHUNK,), jnp.int32),
        pltpu.VMEM((CHUNK,), jnp.float32),
    ],
)
def _deg_kernel(dst_hbm, ones_hbm, zeros_hbm, out_hbm, acc, idx_buf, ones_buf):
    c = lax.axis_index("c")
    s = lax.axis_index("s")
    pltpu.sync_copy(ones_hbm, ones_buf)
    pltpu.sync_copy(
        zeros_hbm.at[pl.ds(s * DEG_STRIPE, DEG_STRIPE)],
        acc.at[pl.ds(s * DEG_STRIPE, DEG_STRIPE)],
    )
    plsc.subcore_barrier()
    w = s * NC + c
    per_w = E_PAD // (NC * NS)
    base = w * per_w

    def step(k, carry):
        pltpu.sync_copy(dst_hbm.at[pl.ds(base + k * CHUNK, CHUNK)], idx_buf)
        pltpu.sync_copy(ones_buf, acc.at[idx_buf], add=True)
        return carry

    lax.fori_loop(0, per_w // CHUNK, step, 0)
    plsc.subcore_barrier()
    pltpu.sync_copy(
        acc.at[pl.ds(s * DEG_STRIPE, DEG_STRIPE)],
        out_hbm.at[c, pl.ds(s * DEG_STRIPE, DEG_STRIPE)],
    )


def _make_agg_kernel(dc):
    rows_per_s = N_ACC // NS
    wb_rows = N // NS

    @functools.partial(
        pl.kernel,
        out_type=(
            jax.ShapeDtypeStruct((N, dc), jnp.float32),
            jax.ShapeDtypeStruct((N, dc), jnp.float32),
        ),
        mesh=_mesh,
        scratch_types=[
            pltpu.VMEM_SHARED((N_ACC, dc), jnp.float32),
            pltpu.VMEM((CHUNK,), jnp.int32),
            pltpu.VMEM((CHUNK,), jnp.int32),
            pltpu.VMEM((CHUNK, dc), jnp.float32),
            pltpu.SemaphoreType.DMA,
        ],
    )
    def agg(g0_hbm, g1_hbm, src_hbm, dst_hbm, zeros_hbm, out0_hbm, out1_hbm,
            acc, src_buf, dst_buf, gbuf, gsem):
        c = lax.axis_index("c")
        s = lax.axis_index("s")
        pltpu.sync_copy(
            zeros_hbm.at[pl.ds(s * rows_per_s, rows_per_s)],
            acc.at[pl.ds(s * rows_per_s, rows_per_s)],
        )
        plsc.subcore_barrier()

        per_s = E_PAD // NS
        base = s * per_s

        def run(g_hbm, out_hbm):
            def step(k, carry):
                off = base + k * CHUNK
                pltpu.sync_copy(src_hbm.at[pl.ds(off, CHUNK)], src_buf)
                pltpu.sync_copy(dst_hbm.at[pl.ds(off, CHUNK)], dst_buf)
                pltpu.async_copy(g_hbm.at[src_buf], gbuf, gsem).wait()
                pltpu.sync_copy(gbuf, acc.at[dst_buf], add=True)
                return carry

            lax.fori_loop(0, per_s // CHUNK, step, 0)
            plsc.subcore_barrier()
            pltpu.sync_copy(
                acc.at[pl.ds(s * wb_rows, wb_rows)],
                out_hbm.at[pl.ds(s * wb_rows, wb_rows)],
            )

        @pl.when(c == 0)
        def _():
            run(g0_hbm, out0_hbm)

        @pl.when(c == 1)
        def _():
            run(g1_hbm, out1_hbm)

    return agg


_agg_hid = _make_agg_kernel(HID // 2)
_agg_out = _make_agg_kernel(OUT // 2)


_R = 2000


def _mm1_body(deg0_ref, deg1_ref, x_ref, w_ref, dinv_ref, g0_ref, g1_ref):
    dinv = lax.rsqrt(deg0_ref[...] + deg1_ref[...] + 1.0)
    h = jnp.dot(x_ref[...], w_ref[...], preferred_element_type=jnp.float32)
    g = h * dinv[:, None]
    dinv_ref[...] = dinv
    g0_ref[...] = g[:, : HID // 2]
    g1_ref[...] = g[:, HID // 2:]


def _mm1_call(deg0, deg1, x, W1):
    grid = (N // _R,)
    return pl.pallas_call(
        _mm1_body,
        grid=grid,
        in_specs=[
            pl.BlockSpec((_R,), lambda i: (i,)),
            pl.BlockSpec((_R,), lambda i: (i,)),
            pl.BlockSpec((_R, IN), lambda i: (i, 0)),
            pl.BlockSpec((IN, HID), lambda i: (0, 0)),
        ],
        out_specs=[
            pl.BlockSpec((_R,), lambda i: (i,)),
            pl.BlockSpec((_R, HID // 2), lambda i: (i, 0)),
            pl.BlockSpec((_R, HID // 2), lambda i: (i, 0)),
        ],
        out_shape=[
            jax.ShapeDtypeStruct((N,), jnp.float32),
            jax.ShapeDtypeStruct((N, HID // 2), jnp.float32),
            jax.ShapeDtypeStruct((N, HID // 2), jnp.float32),
        ],
    )(deg0, deg1, x, W1)


def _mid_body(dinv_ref, s0_ref, s1_ref, g0_ref, g1_ref, b1_ref, w2_ref,
              emb_ref, g2a_ref, g2b_ref):
    dv = dinv_ref[...][:, None]
    b1 = b1_ref[...]
    e0 = (s0_ref[...] + g0_ref[...]) * dv + b1[None, : HID // 2]
    e1 = (s1_ref[...] + g1_ref[...]) * dv + b1[None, HID // 2:]
    emb = jnp.concatenate([e0, e1], axis=1)
    emb_ref[...] = emb
    h = jnp.maximum(emb, 0.0)
    mm2 = jnp.dot(h, w2_ref[...], preferred_element_type=jnp.float32)
    g2 = mm2 * dv
    g2a_ref[...] = g2[:, : OUT // 2]
    g2b_ref[...] = g2[:, OUT // 2:]


def _mid_call(dinv, s0, s1, g0, g1, b1, W2):
    grid = (N // _R,)
    return pl.pallas_call(
        _mid_body,
        grid=grid,
        in_specs=[
            pl.BlockSpec((_R,), lambda i: (i,)),
            pl.BlockSpec((_R, HID // 2), lambda i: (i, 0)),
            pl.BlockSpec((_R, HID // 2), lambda i: (i, 0)),
            pl.BlockSpec((_R, HID // 2), lambda i: (i, 0)),
            pl.BlockSpec((_R, HID // 2), lambda i: (i, 0)),
            pl.BlockSpec((HID,), lambda i: (0,)),
            pl.BlockSpec((HID, OUT), lambda i: (0, 0)),
        ],
        out_specs=[
            pl.BlockSpec((_R, HID), lambda i: (i, 0)),
            pl.BlockSpec((_R, OUT // 2), lambda i: (i, 0)),
            pl.BlockSpec((_R, OUT // 2), lambda i: (i, 0)),
        ],
        out_shape=[
            jax.ShapeDtypeStruct((N, HID), jnp.float32),
            jax.ShapeDtypeStruct((N, OUT // 2), jnp.float32),
            jax.ShapeDtypeStruct((N, OUT // 2), jnp.float32),
        ],
    )(dinv, s0, s1, g0, g1, b1, W2)


def _out_body(dinv_ref, s2a_ref, s2b_ref, g2a_ref, g2b_ref, b2_ref, out_ref):
    dv = dinv_ref[...][:, None]
    b2 = b2_ref[...]
    p0 = (s2a_ref[...] + g2a_ref[...]) * dv + b2[None, : OUT // 2]
    p1 = (s2b_ref[...] + g2b_ref[...]) * dv + b2[None, OUT // 2:]
    p = jnp.concatenate([p0, p1], axis=1)
    m = jnp.max(p, axis=1, keepdims=True)
    lse = jnp.log(jnp.sum(jnp.exp(p - m), axis=1, keepdims=True)) + m
    out_ref[...] = p - lse


def _out_call(dinv, s2a, s2b, g2a, g2b, b2):
    grid = (N // _R,)
    return pl.pallas_call(
        _out_body,
        grid=grid,
        in_specs=[
            pl.BlockSpec((_R,), lambda i: (i,)),
            pl.BlockSpec((_R, OUT // 2), lambda i: (i, 0)),
            pl.BlockSpec((_R, OUT // 2), lambda i: (i, 0)),
            pl.BlockSpec((_R, OUT // 2), lambda i: (i, 0)),
            pl.BlockSpec((_R, OUT // 2), lambda i: (i, 0)),
            pl.BlockSpec((OUT,), lambda i: (0,)),
        ],
        out_specs=pl.BlockSpec((_R, OUT), lambda i: (i, 0)),
        out_shape=jax.ShapeDtypeStruct((N, OUT), jnp.float32),
    )(dinv, s2a, s2b, g2a, g2b, b2)


def kernel(x, edge_index, W1, b1, W2, b2):
    src = edge_index[0]
    dst = edge_index[1]
    npad = E_PAD - E
    ar = jnp.arange(npad, dtype=jnp.int32)
    srcp = jnp.concatenate([src, ar % N])
    dstp = jnp.concatenate([dst, N + (ar % PAD_ROWS)])

    ones_c = jnp.ones((CHUNK,), jnp.float32)
    zeros_deg = jnp.zeros((N_DEG,), jnp.float32)
    zeros_h = jnp.zeros((N_ACC, HID // 2), jnp.float32)
    zeros_o = jnp.zeros((N_ACC, OUT // 2), jnp.float32)

    deg2 = _deg_kernel(dstp, ones_c, zeros_deg)
    dinv, g0, g1 = _mm1_call(deg2[0, :N], deg2[1, :N], x, W1)
    s0, s1 = _agg_hid(g0, g1, srcp, dstp, zeros_h)
    emb, g2a, g2b = _mid_call(dinv, s0, s1, g0, g1, b1, W2)
    s2a, s2b = _agg_out(g2a, g2b, srcp, dstp, zeros_o)
    out = _out_call(dinv, s2a, s2b, g2a, g2b, b2)
    return out, emb


# speedup vs baseline: 13.0527x; 13.0527x over previous
import functools

import jax
import jax.numpy as jnp
from jax import lax
from jax.experimental import pallas as pl
from jax.experimental.pallas import tpu as pltpu
from jax.experimental.pallas import tpu_sc as plsc

N = 10000
E = 320000
IN = 128
HID = 256
OUT = 128

NC = 2
NS = 16
CHUNK = 128
E_PAD = 323584
PAD_ROWS = 64
N_ACC = N + PAD_ROWS
DEG_STRIPE = 632
N_DEG = NS * DEG_STRIPE

_mesh = plsc.VectorSubcoreMesh(core_axis_name="c", subcore_axis_name="s")


@functools.partial(
    pl.kernel,
    out_type=(
        jax.ShapeDtypeStruct((N_DEG,), jnp.float32),
        jax.ShapeDtypeStruct((N_DEG,), jnp.float32),
    ),
    mesh=_mesh,
    scratch_types=[
        pltpu.VMEM_SHARED((N_DEG,), jnp.float32),
        pltpu.VMEM((CHUNK,), jnp.int32),
        pltpu.VMEM((CHUNK,), jnp.float32),
        pltpu.VMEM((DEG_STRIPE,), jnp.float32),
    ],
)
def _deg_kernel(dst_hbm, ones_hbm, zeros_hbm, out0_hbm, out1_hbm, acc, idx_buf,
                ones_buf, stripe_buf):
    c = lax.axis_index("c")
    s = lax.axis_index("s")
    pltpu.sync_copy(ones_hbm, ones_buf)
    pltpu.sync_copy(zeros_hbm, stripe_buf)
    pltpu.sync_copy(stripe_buf, acc.at[pl.ds(s * DEG_STRIPE, DEG_STRIPE)])
    plsc.subcore_barrier()
    w = s * NC + c
    per_w = E_PAD // (NC * NS)
    base = w * per_w

    def step(k, carry):
        pltpu.sync_copy(dst_hbm.at[pl.ds(base + k * CHUNK, CHUNK)], idx_buf)
        pltpu.sync_copy(ones_buf, acc.at[idx_buf], add=True)
        return carry

    lax.fori_loop(0, per_w // CHUNK, step, 0)
    plsc.subcore_barrier()

    pltpu.sync_copy(acc.at[pl.ds(s * DEG_STRIPE, DEG_STRIPE)], stripe_buf)

    @pl.when(c == 0)
    def _():
        pltpu.sync_copy(stripe_buf, out0_hbm.at[pl.ds(s * DEG_STRIPE, DEG_STRIPE)])

    @pl.when(c == 1)
    def _():
        pltpu.sync_copy(stripe_buf, out1_hbm.at[pl.ds(s * DEG_STRIPE, DEG_STRIPE)])


def _make_agg_kernel(dc):
    z_stripe = 632
    z_last = N_ACC - (NS - 1) * z_stripe
    w_stripe = 624
    w_last = N - (NS - 1) * w_stripe

    @functools.partial(
        pl.kernel,
        out_type=(
            jax.ShapeDtypeStruct((N, dc), jnp.float32),
            jax.ShapeDtypeStruct((N, dc), jnp.float32),
        ),
        mesh=_mesh,
        scratch_types=[
            pltpu.VMEM_SHARED((N_ACC, dc), jnp.float32),
            pltpu.VMEM((CHUNK,), jnp.int32),
            pltpu.VMEM((CHUNK,), jnp.int32),
            pltpu.VMEM((CHUNK, dc), jnp.float32),
            pltpu.SemaphoreType.DMA,
        ],
    )
    def agg(g0_hbm, g1_hbm, src_hbm, dst_hbm, zeros_hbm, out0_hbm, out1_hbm,
            acc, src_buf, dst_buf, gbuf, gsem):
        c = lax.axis_index("c")
        s = lax.axis_index("s")
        pltpu.sync_copy(zeros_hbm, gbuf)

        def zero_stripe(r0, nrows):
            nfull, rem = nrows // CHUNK, nrows % CHUNK
            for t in range(nfull):
                pltpu.sync_copy(gbuf, acc.at[pl.ds(r0 + t * CHUNK, CHUNK)])
            if rem:
                pltpu.sync_copy(gbuf.at[pl.ds(0, rem)],
                                acc.at[pl.ds(r0 + nfull * CHUNK, rem)])

        @pl.when(s < NS - 1)
        def _():
            zero_stripe(s * z_stripe, z_stripe)

        @pl.when(s == NS - 1)
        def _():
            zero_stripe((NS - 1) * z_stripe, z_last)

        plsc.subcore_barrier()

        per_s = E_PAD // NS
        base = s * per_s

        def run(g_hbm, out_hbm):
            def step(k, carry):
                off = base + k * CHUNK
                pltpu.sync_copy(src_hbm.at[pl.ds(off, CHUNK)], src_buf)
                pltpu.sync_copy(dst_hbm.at[pl.ds(off, CHUNK)], dst_buf)
                pltpu.async_copy(g_hbm.at[src_buf], gbuf, gsem).wait()
                pltpu.sync_copy(gbuf, acc.at[dst_buf], add=True)
                return carry

            lax.fori_loop(0, per_s // CHUNK, step, 0)
            plsc.subcore_barrier()

            def wb(w0, nrows):
                nfull, rem = nrows // CHUNK, nrows % CHUNK
                for t in range(nfull):
                    pltpu.sync_copy(acc.at[pl.ds(w0 + t * CHUNK, CHUNK)], gbuf)
                    pltpu.sync_copy(gbuf, out_hbm.at[pl.ds(w0 + t * CHUNK, CHUNK)])
                if rem:
                    pltpu.sync_copy(acc.at[pl.ds(w0 + nfull * CHUNK, rem)],
                                    gbuf.at[pl.ds(0, rem)])
                    pltpu.sync_copy(gbuf.at[pl.ds(0, rem)],
                                    out_hbm.at[pl.ds(w0 + nfull * CHUNK, rem)])

            @pl.when(s < NS - 1)
            def _():
                wb(s * w_stripe, w_stripe)

            @pl.when(s == NS - 1)
            def _():
                wb((NS - 1) * w_stripe, w_last)

        @pl.when(c == 0)
        def _():
            run(g0_hbm, out0_hbm)

        @pl.when(c == 1)
        def _():
            run(g1_hbm, out1_hbm)

    return agg


_agg_hid = _make_agg_kernel(HID // 2)


def _make_agg_edgesplit():
    z_stripe = 632
    z_last = N_ACC - (NS - 1) * z_stripe
    w_stripe = 624
    w_last = N - (NS - 1) * w_stripe

    @functools.partial(
        pl.kernel,
        out_type=(
            jax.ShapeDtypeStruct((N, OUT), jnp.float32),
            jax.ShapeDtypeStruct((N, OUT), jnp.float32),
        ),
        mesh=_mesh,
        scratch_types=[
            pltpu.VMEM_SHARED((N_ACC, OUT), jnp.float32),
            pltpu.VMEM((CHUNK,), jnp.int32),
            pltpu.VMEM((CHUNK,), jnp.int32),
            pltpu.VMEM((CHUNK, OUT), jnp.float32),
            pltpu.SemaphoreType.DMA,
        ],
    )
    def agg2(g_hbm, src_hbm, dst_hbm, zeros_hbm, out0_hbm, out1_hbm,
             acc, src_buf, dst_buf, gbuf, gsem):
        c = lax.axis_index("c")
        s = lax.axis_index("s")
        pltpu.sync_copy(zeros_hbm, gbuf)

        def zero_stripe(r0, nrows):
            nfull, rem = nrows // CHUNK, nrows % CHUNK
            for t in range(nfull):
                pltpu.sync_copy(gbuf, acc.at[pl.ds(r0 + t * CHUNK, CHUNK)])
            if rem:
                pltpu.sync_copy(gbuf.at[pl.ds(0, rem)],
                                acc.at[pl.ds(r0 + nfull * CHUNK, rem)])

        @pl.when(s < NS - 1)
        def _():
            zero_stripe(s * z_stripe, z_stripe)

        @pl.when(s == NS - 1)
        def _():
            zero_stripe((NS - 1) * z_stripe, z_last)

        plsc.subcore_barrier()

        per_w = E_PAD // (NC * NS)
        base = c * (E_PAD // NC) + s * per_w

        def step(k, carry):
            off = base + k * CHUNK
            pltpu.sync_copy(src_hbm.at[pl.ds(off, CHUNK)], src_buf)
            pltpu.sync_copy(dst_hbm.at[pl.ds(off, CHUNK)], dst_buf)
            pltpu.async_copy(g_hbm.at[src_buf], gbuf, gsem).wait()
            pltpu.sync_copy(gbuf, acc.at[dst_buf], add=True)
            return carry

        lax.fori_loop(0, per_w // CHUNK, step, 0)
        plsc.subcore_barrier()

        def wb(out_hbm, w0, nrows):
            nfull, rem = nrows // CHUNK, nrows % CHUNK
            for t in range(nfull):
                pltpu.sync_copy(acc.at[pl.ds(w0 + t * CHUNK, CHUNK)], gbuf)
                pltpu.sync_copy(gbuf, out_hbm.at[pl.ds(w0 + t * CHUNK, CHUNK)])
            if rem:
                pltpu.sync_copy(acc.at[pl.ds(w0 + nfull * CHUNK, rem)],
                                gbuf.at[pl.ds(0, rem)])
                pltpu.sync_copy(gbuf.at[pl.ds(0, rem)],
                                out_hbm.at[pl.ds(w0 + nfull * CHUNK, rem)])

        def wb_all(out_hbm):
            @pl.when(s < NS - 1)
            def _():
                wb(out_hbm, s * w_stripe, w_stripe)

            @pl.when(s == NS - 1)
            def _():
                wb(out_hbm, (NS - 1) * w_stripe, w_last)

        @pl.when(c == 0)
        def _():
            wb_all(out0_hbm)

        @pl.when(c == 1)
        def _():
            wb_all(out1_hbm)

    return agg2


_agg_out = _make_agg_edgesplit()


_R = 2000


def _mm1_body(deg0_ref, deg1_ref, x_ref, w_ref, dinv_ref, g0_ref, g1_ref):
    dinv = lax.rsqrt(deg0_ref[...] + deg1_ref[...] + 1.0)
    h = jnp.dot(x_ref[...], w_ref[...], preferred_element_type=jnp.float32)
    g = h * dinv
    dinv_ref[...] = dinv
    g0_ref[...] = g[:, : HID // 2]
    g1_ref[...] = g[:, HID // 2:]


def _mm1_call(deg0, deg1, x, W1):
    grid = (N // _R,)
    return pl.pallas_call(
        _mm1_body,
        grid=grid,
        in_specs=[
            pl.BlockSpec((_R, 1), lambda i: (i, 0)),
            pl.BlockSpec((_R, 1), lambda i: (i, 0)),
            pl.BlockSpec((_R, IN), lambda i: (i, 0)),
            pl.BlockSpec((IN, HID), lambda i: (0, 0)),
        ],
        out_specs=[
            pl.BlockSpec((_R, 1), lambda i: (i, 0)),
            pl.BlockSpec((_R, HID // 2), lambda i: (i, 0)),
            pl.BlockSpec((_R, HID // 2), lambda i: (i, 0)),
        ],
        out_shape=[
            jax.ShapeDtypeStruct((N, 1), jnp.float32),
            jax.ShapeDtypeStruct((N, HID // 2), jnp.float32),
            jax.ShapeDtypeStruct((N, HID // 2), jnp.float32),
        ],
    )(deg0, deg1, x, W1)


def _mid_body(dinv_ref, s0_ref, s1_ref, g0_ref, g1_ref, b1_ref, w2_ref,
              emb_ref, g2_ref):
    dv = dinv_ref[...]
    b1 = b1_ref[...]
    e0 = (s0_ref[...] + g0_ref[...]) * dv + b1[None, : HID // 2]
    e1 = (s1_ref[...] + g1_ref[...]) * dv + b1[None, HID // 2:]
    emb = jnp.concatenate([e0, e1], axis=1)
    emb_ref[...] = emb
    h = jnp.maximum(emb, 0.0)
    mm2 = jnp.dot(h, w2_ref[...], preferred_element_type=jnp.float32)
    g2_ref[...] = mm2 * dv


def _mid_call(dinv, s0, s1, g0, g1, b1, W2):
    grid = (N // _R,)
    return pl.pallas_call(
        _mid_body,
        grid=grid,
        in_specs=[
            pl.BlockSpec((_R, 1), lambda i: (i, 0)),
            pl.BlockSpec((_R, HID // 2), lambda i: (i, 0)),
            pl.BlockSpec((_R, HID // 2), lambda i: (i, 0)),
            pl.BlockSpec((_R, HID // 2), lambda i: (i, 0)),
            pl.BlockSpec((_R, HID // 2), lambda i: (i, 0)),
            pl.BlockSpec((HID,), lambda i: (0,)),
            pl.BlockSpec((HID, OUT), lambda i: (0, 0)),
        ],
        out_specs=[
            pl.BlockSpec((_R, HID), lambda i: (i, 0)),
            pl.BlockSpec((_R, OUT), lambda i: (i, 0)),
        ],
        out_shape=[
            jax.ShapeDtypeStruct((N, HID), jnp.float32),
            jax.ShapeDtypeStruct((N, OUT), jnp.float32),
        ],
    )(dinv, s0, s1, g0, g1, b1, W2)


def _out_body(dinv_ref, s2a_ref, s2b_ref, g2_ref, b2_ref, out_ref):
    dv = dinv_ref[...]
    b2 = b2_ref[...]
    p = (s2a_ref[...] + s2b_ref[...] + g2_ref[...]) * dv + b2[None, :]
    m = jnp.max(p, axis=1, keepdims=True)
    lse = jnp.log(jnp.sum(jnp.exp(p - m), axis=1, keepdims=True)) + m
    out_ref[...] = p - lse


def _out_call(dinv, s2a, s2b, g2, b2):
    grid = (N // _R,)
    return pl.pallas_call(
        _out_body,
        grid=grid,
        in_specs=[
            pl.BlockSpec((_R, 1), lambda i: (i, 0)),
            pl.BlockSpec((_R, OUT), lambda i: (i, 0)),
            pl.BlockSpec((_R, OUT), lambda i: (i, 0)),
            pl.BlockSpec((_R, OUT), lambda i: (i, 0)),
            pl.BlockSpec((OUT,), lambda i: (0,)),
        ],
        out_specs=pl.BlockSpec((_R, OUT), lambda i: (i, 0)),
        out_shape=jax.ShapeDtypeStruct((N, OUT), jnp.float32),
    )(dinv, s2a, s2b, g2, b2)


def kernel(x, edge_index, W1, b1, W2, b2):
    src = edge_index[0]
    dst = edge_index[1]
    npad = E_PAD - E
    ar = jnp.arange(npad, dtype=jnp.int32)
    srcp = jnp.concatenate([src, ar % N])
    dstp = jnp.concatenate([dst, N + (ar % PAD_ROWS)])

    ones_c = jnp.ones((CHUNK,), jnp.float32)
    zeros_deg = jnp.zeros((DEG_STRIPE,), jnp.float32)
    zeros_h = jnp.zeros((CHUNK, HID // 2), jnp.float32)
    zeros_o = jnp.zeros((CHUNK, OUT), jnp.float32)

    dega, degb = _deg_kernel(dstp, ones_c, zeros_deg)
    dinv, g0, g1 = _mm1_call(dega[:N, None], degb[:N, None], x, W1)
    s0, s1 = _agg_hid(g0, g1, srcp, dstp, zeros_h)
    emb, g2 = _mid_call(dinv, s0, s1, g0, g1, b1, W2)
    s2a, s2b = _agg_out(g2, srcp, dstp, zeros_o)
    out = _out_call(dinv, s2a, s2b, g2, b2)
    return out, emb


# speedup vs baseline: 26.0790x; 1.9980x over previous
import functools

import jax
import jax.numpy as jnp
from jax import lax
from jax.experimental import pallas as pl
from jax.experimental.pallas import tpu as pltpu
from jax.experimental.pallas import tpu_sc as plsc

N = 10000
E = 320000
IN = 128
HID = 256
OUT = 128

NC = 2
NS = 16
CHUNK = 128
NCH = 2560
E_PAD = NCH * CHUNK
PAD_ROWS = 64
N_ACC = N + PAD_ROWS
DEG_STRIPE = 632
N_DEG = NS * DEG_STRIPE

K1 = NCH // NS
K2 = NCH // (NC * NS)
IB1 = 32
IB2 = 16

_mesh = plsc.VectorSubcoreMesh(core_axis_name="c", subcore_axis_name="s")


@functools.partial(
    pl.kernel,
    out_type=(
        jax.ShapeDtypeStruct((N_DEG,), jnp.float32),
        jax.ShapeDtypeStruct((N_DEG,), jnp.float32),
    ),
    mesh=_mesh,
    scratch_types=[
        pltpu.VMEM_SHARED((N_DEG,), jnp.float32),
        pltpu.VMEM((K2, CHUNK), jnp.int32),
        pltpu.VMEM((CHUNK,), jnp.float32),
        pltpu.VMEM((DEG_STRIPE,), jnp.float32),
    ],
)
def _deg_kernel(dst_hbm, ones_hbm, zeros_hbm, out0_hbm, out1_hbm, acc, idx_v,
                ones_buf, stripe_buf):
    c = lax.axis_index("c")
    s = lax.axis_index("s")
    w = s * NC + c
    pltpu.sync_copy(dst_hbm.at[pl.ds(w * K2, K2)], idx_v)
    pltpu.sync_copy(ones_hbm, ones_buf)
    pltpu.sync_copy(zeros_hbm, stripe_buf)
    pltpu.sync_copy(stripe_buf, acc.at[pl.ds(s * DEG_STRIPE, DEG_STRIPE)])
    plsc.subcore_barrier()

    def step(k, carry):
        pltpu.sync_copy(ones_buf, acc.at[idx_v.at[k]], add=True)
        return carry

    lax.fori_loop(0, K2, step, 0)
    plsc.subcore_barrier()

    pltpu.sync_copy(acc.at[pl.ds(s * DEG_STRIPE, DEG_STRIPE)], stripe_buf)

    @pl.when(c == 0)
    def _():
        pltpu.sync_copy(stripe_buf, out0_hbm.at[pl.ds(s * DEG_STRIPE, DEG_STRIPE)])

    @pl.when(c == 1)
    def _():
        pltpu.sync_copy(stripe_buf, out1_hbm.at[pl.ds(s * DEG_STRIPE, DEG_STRIPE)])


_Z_STRIPE = 632
_Z_LAST = N_ACC - (NS - 1) * _Z_STRIPE
_W_STRIPE = 624
_W_LAST = N - (NS - 1) * _W_STRIPE


def _zero_acc(s, acc, zeros_hbm, gbuf):
    pltpu.sync_copy(zeros_hbm, gbuf)

    def zero_stripe(r0, nrows):
        nfull, rem = nrows // CHUNK, nrows % CHUNK
        for t in range(nfull):
            pltpu.sync_copy(gbuf, acc.at[pl.ds(r0 + t * CHUNK, CHUNK)])
        if rem:
            pltpu.sync_copy(gbuf.at[pl.ds(0, rem)],
                            acc.at[pl.ds(r0 + nfull * CHUNK, rem)])

    @pl.when(s < NS - 1)
    def _():
        zero_stripe(s * _Z_STRIPE, _Z_STRIPE)

    @pl.when(s == NS - 1)
    def _():
        zero_stripe((NS - 1) * _Z_STRIPE, _Z_LAST)


def _writeback(s, acc, out_hbm, gbuf):
    def wb(w0, nrows):
        nfull, rem = nrows // CHUNK, nrows % CHUNK
        for t in range(nfull):
            pltpu.sync_copy(acc.at[pl.ds(w0 + t * CHUNK, CHUNK)], gbuf)
            pltpu.sync_copy(gbuf, out_hbm.at[pl.ds(w0 + t * CHUNK, CHUNK)])
        if rem:
            pltpu.sync_copy(acc.at[pl.ds(w0 + nfull * CHUNK, rem)],
                            gbuf.at[pl.ds(0, rem)])
            pltpu.sync_copy(gbuf.at[pl.ds(0, rem)],
                            out_hbm.at[pl.ds(w0 + nfull * CHUNK, rem)])

    @pl.when(s < NS - 1)
    def _():
        wb(s * _W_STRIPE, _W_STRIPE)

    @pl.when(s == NS - 1)
    def _():
        wb((NS - 1) * _W_STRIPE, _W_LAST)


def _agg_loop(nb, ib, row0, src_hbm, dst_hbm, g_hbm, acc, src_v, dst_v,
              gbuf0, gbuf1, sem0, sem1):
    def blk(j, carry):
        r = row0 + j * ib
        pltpu.sync_copy(src_hbm.at[pl.ds(r, ib)], src_v)
        pltpu.sync_copy(dst_hbm.at[pl.ds(r, ib)], dst_v)

        def gdesc(k, gbuf, sem):
            return pltpu.make_async_copy(g_hbm.at[src_v.at[k]], gbuf, sem)

        gdesc(0, gbuf0, sem0).start()
        gdesc(1, gbuf1, sem1).start()

        def body(k, gbuf, sem):
            gdesc(k, gbuf, sem).wait()
            pltpu.sync_copy(gbuf, acc.at[dst_v.at[k]], add=True)

            @pl.when(k + 2 < ib)
            def _():
                gdesc(k + 2, gbuf, sem).start()

        def step(k, c2):
            @pl.when(lax.rem(k, 2) == 0)
            def _():
                body(k, gbuf0, sem0)

            @pl.when(lax.rem(k, 2) == 1)
            def _():
                body(k, gbuf1, sem1)

            return c2

        lax.fori_loop(0, ib, step, 0)
        return carry

    lax.fori_loop(0, nb, blk, 0)


@functools.partial(
    pl.kernel,
    out_type=(
        jax.ShapeDtypeStruct((N, HID // 2), jnp.float32),
        jax.ShapeDtypeStruct((N, HID // 2), jnp.float32),
    ),
    mesh=_mesh,
    scratch_types=[
        pltpu.VMEM_SHARED((N_ACC, HID // 2), jnp.float32),
        pltpu.VMEM((IB1, CHUNK), jnp.int32),
        pltpu.VMEM((IB1, CHUNK), jnp.int32),
        pltpu.VMEM((CHUNK, HID // 2), jnp.float32),
        pltpu.VMEM((CHUNK, HID // 2), jnp.float32),
        pltpu.SemaphoreType.DMA,
        pltpu.SemaphoreType.DMA,
    ],
)
def _agg_hid(g0_hbm, g1_hbm, src_hbm, dst_hbm, zeros_hbm, out0_hbm, out1_hbm,
             acc, src_v, dst_v, gbuf0, gbuf1, sem0, sem1):
    c = lax.axis_index("c")
    s = lax.axis_index("s")
    _zero_acc(s, acc, zeros_hbm, gbuf0)
    plsc.subcore_barrier()

    def run(g_hbm, out_hbm):
        _agg_loop(K1 // IB1, IB1, s * K1, src_hbm, dst_hbm, g_hbm, acc,
                  src_v, dst_v, gbuf0, gbuf1, sem0, sem1)
        plsc.subcore_barrier()
        _writeback(s, acc, out_hbm, gbuf0)

    @pl.when(c == 0)
    def _():
        run(g0_hbm, out0_hbm)

    @pl.when(c == 1)
    def _():
        run(g1_hbm, out1_hbm)


@functools.partial(
    pl.kernel,
    out_type=(
        jax.ShapeDtypeStruct((N, OUT), jnp.float32),
        jax.ShapeDtypeStruct((N, OUT), jnp.float32),
    ),
    mesh=_mesh,
    scratch_types=[
        pltpu.VMEM_SHARED((N_ACC, OUT), jnp.float32),
        pltpu.VMEM((IB2, CHUNK), jnp.int32),
        pltpu.VMEM((IB2, CHUNK), jnp.int32),
        pltpu.VMEM((CHUNK, OUT), jnp.float32),
        pltpu.VMEM((CHUNK, OUT), jnp.float32),
        pltpu.SemaphoreType.DMA,
        pltpu.SemaphoreType.DMA,
    ],
)
def _agg_out(g_hbm, src_hbm, dst_hbm, zeros_hbm, out0_hbm, out1_hbm,
             acc, src_v, dst_v, gbuf0, gbuf1, sem0, sem1):
    c = lax.axis_index("c")
    s = lax.axis_index("s")
    row0 = c * (NCH // NC) + s * K2
    _zero_acc(s, acc, zeros_hbm, gbuf0)
    plsc.subcore_barrier()

    _agg_loop(K2 // IB2, IB2, row0, src_hbm, dst_hbm, g_hbm, acc,
              src_v, dst_v, gbuf0, gbuf1, sem0, sem1)
    plsc.subcore_barrier()

    @pl.when(c == 0)
    def _():
        _writeback(s, acc, out0_hbm, gbuf0)

    @pl.when(c == 1)
    def _():
        _writeback(s, acc, out1_hbm, gbuf0)


_R = 2000


def _mm1_body(deg0_ref, deg1_ref, x_ref, w_ref, dinv_ref, g0_ref, g1_ref):
    dinv = lax.rsqrt(deg0_ref[...] + deg1_ref[...] + 1.0)
    h = jnp.dot(x_ref[...], w_ref[...], preferred_element_type=jnp.float32)
    g = h * dinv
    dinv_ref[...] = dinv
    g0_ref[...] = g[:, : HID // 2]
    g1_ref[...] = g[:, HID // 2:]


def _mm1_call(deg0, deg1, x, W1):
    grid = (N // _R,)
    return pl.pallas_call(
        _mm1_body,
        grid=grid,
        in_specs=[
            pl.BlockSpec((_R, 1), lambda i: (i, 0)),
            pl.BlockSpec((_R, 1), lambda i: (i, 0)),
            pl.BlockSpec((_R, IN), lambda i: (i, 0)),
            pl.BlockSpec((IN, HID), lambda i: (0, 0)),
        ],
        out_specs=[
            pl.BlockSpec((_R, 1), lambda i: (i, 0)),
            pl.BlockSpec((_R, HID // 2), lambda i: (i, 0)),
            pl.BlockSpec((_R, HID // 2), lambda i: (i, 0)),
        ],
        out_shape=[
            jax.ShapeDtypeStruct((N, 1), jnp.float32),
            jax.ShapeDtypeStruct((N, HID // 2), jnp.float32),
            jax.ShapeDtypeStruct((N, HID // 2), jnp.float32),
        ],
    )(deg0, deg1, x, W1)


def _mid_body(dinv_ref, s0_ref, s1_ref, g0_ref, g1_ref, b1_ref, w2_ref,
              emb_ref, g2_ref):
    dv = dinv_ref[...]
    b1 = b1_ref[...]
    e0 = (s0_ref[...] + g0_ref[...]) * dv + b1[None, : HID // 2]
    e1 = (s1_ref[...] + g1_ref[...]) * dv + b1[None, HID // 2:]
    emb = jnp.concatenate([e0, e1], axis=1)
    emb_ref[...] = emb
    h = jnp.maximum(emb, 0.0)
    mm2 = jnp.dot(h, w2_ref[...], preferred_element_type=jnp.float32)
    g2_ref[...] = mm2 * dv


def _mid_call(dinv, s0, s1, g0, g1, b1, W2):
    grid = (N // _R,)
    return pl.pallas_call(
        _mid_body,
        grid=grid,
        in_specs=[
            pl.BlockSpec((_R, 1), lambda i: (i, 0)),
            pl.BlockSpec((_R, HID // 2), lambda i: (i, 0)),
            pl.BlockSpec((_R, HID // 2), lambda i: (i, 0)),
            pl.BlockSpec((_R, HID // 2), lambda i: (i, 0)),
            pl.BlockSpec((_R, HID // 2), lambda i: (i, 0)),
            pl.BlockSpec((HID,), lambda i: (0,)),
            pl.BlockSpec((HID, OUT), lambda i: (0, 0)),
        ],
        out_specs=[
            pl.BlockSpec((_R, HID), lambda i: (i, 0)),
            pl.BlockSpec((_R, OUT), lambda i: (i, 0)),
        ],
        out_shape=[
            jax.ShapeDtypeStruct((N, HID), jnp.float32),
            jax.ShapeDtypeStruct((N, OUT), jnp.float32),
        ],
    )(dinv, s0, s1, g0, g1, b1, W2)


def _out_body(dinv_ref, s2a_ref, s2b_ref, g2_ref, b2_ref, out_ref):
    dv = dinv_ref[...]
    b2 = b2_ref[...]
    p = (s2a_ref[...] + s2b_ref[...] + g2_ref[...]) * dv + b2[None, :]
    m = jnp.max(p, axis=1, keepdims=True)
    lse = jnp.log(jnp.sum(jnp.exp(p - m), axis=1, keepdims=True)) + m
    out_ref[...] = p - lse


def _out_call(dinv, s2a, s2b, g2, b2):
    grid = (N // _R,)
    return pl.pallas_call(
        _out_body,
        grid=grid,
        in_specs=[
            pl.BlockSpec((_R, 1), lambda i: (i, 0)),
            pl.BlockSpec((_R, OUT), lambda i: (i, 0)),
            pl.BlockSpec((_R, OUT), lambda i: (i, 0)),
            pl.BlockSpec((_R, OUT), lambda i: (i, 0)),
            pl.BlockSpec((OUT,), lambda i: (0,)),
        ],
        out_specs=pl.BlockSpec((_R, OUT), lambda i: (i, 0)),
        out_shape=jax.ShapeDtypeStruct((N, OUT), jnp.float32),
    )(dinv, s2a, s2b, g2, b2)


def kernel(x, edge_index, W1, b1, W2, b2):
    src = edge_index[0]
    dst = edge_index[1]
    npad = E_PAD - E
    ar = jnp.arange(npad, dtype=jnp.int32)
    srcp = jnp.concatenate([src, ar % N]).reshape(NCH, CHUNK)
    dstp = jnp.concatenate([dst, N + (ar % PAD_ROWS)]).reshape(NCH, CHUNK)

    ones_c = jnp.ones((CHUNK,), jnp.float32)
    zeros_deg = jnp.zeros((DEG_STRIPE,), jnp.float32)
    zeros_h = jnp.zeros((CHUNK, HID // 2), jnp.float32)
    zeros_o = jnp.zeros((CHUNK, OUT), jnp.float32)

    dega, degb = _deg_kernel(dstp, ones_c, zeros_deg)
    dinv, g0, g1 = _mm1_call(dega[:N, None], degb[:N, None], x, W1)
    s0, s1 = _agg_hid(g0, g1, srcp, dstp, zeros_h)
    emb, g2 = _mid_call(dinv, s0, s1, g0, g1, b1, W2)
    s2a, s2b = _agg_out(g2, srcp, dstp, zeros_o)
    out = _out_call(dinv, s2a, s2b, g2, b2)
    return out, emb
